# baseline (speedup 1.0000x reference)
import functools

import jax
import jax.numpy as jnp
from jax import lax
from jax.experimental import pallas as pl
from jax.experimental.pallas import tpu as pltpu
from jax.experimental.pallas import tpu_sc as plsc

NC = 2
NS = 16
NW = NC * NS
LANES = 16

BATCH = 4
SEQ = 8192
DIM = 1024
ROWS_W = SEQ // NW
T = 16
CHUNKS = ROWS_W // T
NPASS = CHUNKS * BATCH


def _sc_body(x_hbm, pos_hbm, out_hbm,
             xb0, xb1, xb2, xb3, pb0, pb1,
             xi0, xi1, xi2, xi3, xo0, xo1, xo2, xo3, ps0, ps1):
    xb = [xb0, xb1, xb2, xb3]
    pb = [pb0, pb1]
    xisem = [xi0, xi1, xi2, xi3]
    xosem = [xo0, xo1, xo2, xo3]
    psem = [ps0, ps1]

    wid = lax.axis_index("s") * NC + lax.axis_index("c")
    base = wid * ROWS_W
    last_ci = CHUNKS - 1

    def x_row(ci, b):
        return b * SEQ + base + ci * T

    def issue_x_in(ci, b, s):
        pltpu.async_copy(
            x_hbm.at[pl.ds(x_row(ci, b), T)], xb[s], xisem[s])

    def issue_x_out(ci, b, s):
        pltpu.async_copy(
            xb[s], out_hbm.at[pl.ds(x_row(ci, b), T)], xosem[s])

    def issue_pos(ci, q):
        pltpu.async_copy(
            pos_hbm.at[pl.ds(base + ci * T, T)], pb[q], psem[q])

    def wait_x_in(s):
        pltpu.make_async_copy(
            x_hbm.at[pl.ds(0, T)], xb[s], xisem[s]).wait()

    def wait_x_out(s):
        pltpu.make_async_copy(
            xb[s], out_hbm.at[pl.ds(0, T)], xosem[s]).wait()

    def wait_pos(q):
        pltpu.make_async_copy(
            pos_hbm.at[pl.ds(0, T)], pb[q], psem[q]).wait()

    issue_pos(0, 0)
    issue_x_in(0, 0, 0)
    issue_x_in(0, 1, 1)

    @pl.loop(0, CHUNKS, step=2)
    def _(ci0):
        for q in (0, 1):
            ci = ci0 + q
            ci_next = jnp.minimum(ci + 1, last_ci)
            wait_pos(q)
            issue_pos(ci_next, 1 - q)
            for b in range(BATCH):
                s = b
                s2 = (b + 2) % 4
                if q == 0 and b < 2:
                    @pl.when(ci0 > 0)
                    def _():
                        wait_x_out(s2)
                else:
                    wait_x_out(s2)
                if b < 2:
                    issue_x_in(ci, b + 2, s2)
                else:
                    issue_x_in(ci_next, b - 2, s2)
                wait_x_in(s)
                pbuf = pb[q]
                xbuf = xb[s]

                @plsc.parallel_loop(0, T, 1)
                def _(r):
                    for j in range(DIM // LANES):
                        c = j * LANES
                        xbuf[r, pl.ds(c, LANES)] = (
                            xbuf[r, pl.ds(c, LANES)]
                            + pbuf[r, pl.ds(c, LANES)]
                        )

                issue_x_out(ci, b, s)

    wait_x_out((NPASS - 2) % 4)
    wait_x_out((NPASS - 1) % 4)
    wait_x_in(NPASS % 4)
    wait_x_in((NPASS + 1) % 4)
    wait_pos(CHUNKS % 2)


_sc_call = functools.partial(
    pl.kernel,
    out_type=jax.ShapeDtypeStruct((BATCH * SEQ, DIM), jnp.float32),
    mesh=plsc.VectorSubcoreMesh(core_axis_name="c", subcore_axis_name="s"),
    scratch_types=(
        [pltpu.VMEM((T, DIM), jnp.float32) for _ in range(6)]
        + [pltpu.SemaphoreType.DMA for _ in range(10)]
    ),
)(_sc_body)


def kernel(x, pos_table):
    batch, seq_len, dim = x.shape
    out = _sc_call(x.reshape(batch * seq_len, dim), pos_table[:seq_len])
    return out.reshape(x.shape)
